# baseline (speedup 1.0000x reference)
import functools
import math

import jax
import jax.numpy as jnp
from jax import lax
from jax.experimental import pallas as pl
from jax.experimental.pallas import tpu as pltpu
from jax.experimental.pallas import tpu_sc as plsc

D_MODEL = 128
SCALE = math.sqrt(float(D_MODEL))

NUM_CORES = 2
NUM_SUBCORES = 16
NUM_WORKERS = NUM_CORES * NUM_SUBCORES
LANES = 16

B_ROWS = 1024
SEQ = 200
ROWS_PER_W = B_ROWS // NUM_WORKERS

CHUNK = SEQ
SUBWIDTHS = (104, 96)
SUBW = SUBWIDTHS[0]
NCHUNK = ROWS_PER_W
RING = 4
AHEAD = 2
UNROLL_ROWS = 4


def _emb_kernel(idx_hbm, table_hbm, out_hbm, idx2_v, idx_v, *rest):
    bufs = rest[0:RING]
    gsems = rest[RING:2 * RING]
    wsems = rest[2 * RING:3 * RING]

    wid = lax.axis_index("s") * NUM_CORES + lax.axis_index("c")
    row0 = wid * ROWS_PER_W

    pltpu.sync_copy(idx_hbm.at[pl.ds(row0, ROWS_PER_W)], idx2_v)

    col_offs = tuple(range(0, SEQ - LANES, LANES)) + (SEQ - LANES,)

    def repack_body(r, c):
        for off in col_offs:
            idx_v[pl.ds(r * SEQ + off, LANES)] = idx2_v[r, pl.ds(off, LANES)]
        return c

    lax.fori_loop(0, ROWS_PER_W, repack_body, 0)

    def gathers(g, b):
        return [
            pltpu.make_async_copy(
                table_hbm.at[idx_v.at[pl.ds(g * CHUNK + s * SUBW, w)]],
                bufs[b].at[pl.ds(s * SUBW, w)],
                gsems[b])
            for s, w in enumerate(SUBWIDTHS)
        ]

    def write(g, b):
        return pltpu.make_async_copy(bufs[b], out_hbm.at[row0 + g], wsems[b])

    def scale(buf):
        def row_body(i, c):
            for r in range(UNROLL_ROWS):
                row = i * UNROLL_ROWS + r
                for j in range(D_MODEL // LANES):
                    sl = pl.ds(j * LANES, LANES)
                    buf[row, sl] = buf[row, sl] * SCALE
            return c

        lax.fori_loop(0, CHUNK // UNROLL_ROWS, row_body, 0)

    def chunk_body(g, slot, with_start, with_drain):
        nslot = (slot + AHEAD) % RING
        if with_drain:
            write(g - (RING - AHEAD), nslot).wait()
        if with_start:
            for h in gathers(g + AHEAD, nslot):
                h.start()
        for h in gathers(g, slot):
            h.wait()
        scale(bufs[slot])
        write(g, slot).start()

    for c in range(AHEAD):
        for h in gathers(c, c % RING):
            h.start()

    for g in range(RING - AHEAD):
        chunk_body(g, g % RING, with_start=True, with_drain=False)

    g0 = RING - AHEAD
    steady = NCHUNK - (RING - AHEAD) - AHEAD
    rounds, leftover = divmod(steady, RING)

    def outer(t, carry):
        for b in range(RING):
            g = g0 + t * RING + b
            chunk_body(g, (g0 + b) % RING, with_start=True, with_drain=True)
        return carry

    lax.fori_loop(0, rounds, outer, 0)

    for i in range(leftover):
        g = g0 + rounds * RING + i
        chunk_body(g, g % RING, with_start=True, with_drain=True)

    for g in range(NCHUNK - AHEAD, NCHUNK):
        chunk_body(g, g % RING, with_start=False, with_drain=False)

    for g in range(NCHUNK - RING, NCHUNK):
        write(g, g % RING).wait()


@functools.partial(jax.jit, static_argnames=())
def kernel(x, table):
    mesh = plsc.VectorSubcoreMesh(core_axis_name="c", subcore_axis_name="s")
    out = pl.kernel(
        _emb_kernel,
        mesh=mesh,
        out_type=jax.ShapeDtypeStruct((B_ROWS, SEQ, D_MODEL), jnp.float32),
        scratch_types=(
            [pltpu.VMEM((ROWS_PER_W, SEQ), jnp.int32),
             pltpu.VMEM((ROWS_PER_W * SEQ,), jnp.int32)]
            + [pltpu.VMEM((CHUNK, D_MODEL), jnp.float32) for _ in range(RING)]
            + [pltpu.SemaphoreType.DMA for _ in range(2 * RING)]
        ),
    )(x, table)
    return out


# speedup vs baseline: 1.0001x; 1.0001x over previous
import functools
import math

import jax
import jax.numpy as jnp
from jax import lax
from jax.experimental import pallas as pl
from jax.experimental.pallas import tpu as pltpu
from jax.experimental.pallas import tpu_sc as plsc

D_MODEL = 128
SCALE = math.sqrt(float(D_MODEL))

NUM_CORES = 2
NUM_SUBCORES = 16
NUM_WORKERS = NUM_CORES * NUM_SUBCORES
LANES = 16

B_ROWS = 1024
SEQ = 200
ROWS_PER_W = B_ROWS // NUM_WORKERS

B_PER_W = ROWS_PER_W * SEQ
CHUNK = 128
NCHUNK = B_PER_W // CHUNK
RING = 5
AHEAD = 3
UNROLL_ROWS = 4


def _emb_kernel(idx_hbm, table_hbm, out_hbm, idx2_v, idx_v, *rest):
    bufs = rest[0:RING]
    gsems = rest[RING:2 * RING]
    wsems = rest[2 * RING:3 * RING]

    wid = lax.axis_index("s") * NUM_CORES + lax.axis_index("c")
    row0 = wid * ROWS_PER_W

    pltpu.sync_copy(idx_hbm.at[pl.ds(row0, ROWS_PER_W)], idx2_v)

    col_offs = tuple(range(0, SEQ - LANES, LANES)) + (SEQ - LANES,)

    def repack_body(r, c):
        for off in col_offs:
            idx_v[pl.ds(r * SEQ + off, LANES)] = idx2_v[r, pl.ds(off, LANES)]
        return c

    lax.fori_loop(0, ROWS_PER_W, repack_body, 0)

    out_chunk0 = wid * NCHUNK

    def gathers(g, b):
        return [
            pltpu.make_async_copy(
                table_hbm.at[idx_v.at[pl.ds(g * CHUNK, CHUNK)]],
                bufs[b],
                gsems[b])
        ]

    def write(g, b):
        r0 = (out_chunk0 + g) * CHUNK
        return pltpu.make_async_copy(bufs[b], out_hbm.at[pl.ds(r0, CHUNK)],
                                     wsems[b])

    def scale(buf):
        def row_body(i, c):
            for r in range(UNROLL_ROWS):
                row = i * UNROLL_ROWS + r
                for j in range(D_MODEL // LANES):
                    sl = pl.ds(j * LANES, LANES)
                    buf[row, sl] = buf[row, sl] * SCALE
            return c

        lax.fori_loop(0, CHUNK // UNROLL_ROWS, row_body, 0)

    def chunk_body(g, slot, with_start, with_drain):
        nslot = (slot + AHEAD) % RING
        if with_drain:
            write(g - (RING - AHEAD), nslot).wait()
        if with_start:
            for h in gathers(g + AHEAD, nslot):
                h.start()
        for h in gathers(g, slot):
            h.wait()
        scale(bufs[slot])
        write(g, slot).start()

    for c in range(AHEAD):
        for h in gathers(c, c % RING):
            h.start()

    for g in range(RING - AHEAD):
        chunk_body(g, g % RING, with_start=True, with_drain=False)

    g0 = RING - AHEAD
    steady = NCHUNK - (RING - AHEAD) - AHEAD
    rounds, leftover = divmod(steady, RING)

    def outer(t, carry):
        for b in range(RING):
            g = g0 + t * RING + b
            chunk_body(g, (g0 + b) % RING, with_start=True, with_drain=True)
        return carry

    lax.fori_loop(0, rounds, outer, 0)

    for i in range(leftover):
        g = g0 + rounds * RING + i
        chunk_body(g, g % RING, with_start=True, with_drain=True)

    for g in range(NCHUNK - AHEAD, NCHUNK):
        chunk_body(g, g % RING, with_start=False, with_drain=False)

    for g in range(NCHUNK - RING, NCHUNK):
        write(g, g % RING).wait()


@functools.partial(jax.jit, static_argnames=())
def kernel(x, table):
    mesh = plsc.VectorSubcoreMesh(core_axis_name="c", subcore_axis_name="s")
    out = pl.kernel(
        _emb_kernel,
        mesh=mesh,
        out_type=jax.ShapeDtypeStruct((B_ROWS * SEQ, D_MODEL), jnp.float32),
        scratch_types=(
            [pltpu.VMEM((ROWS_PER_W, SEQ), jnp.int32),
             pltpu.VMEM((ROWS_PER_W * SEQ,), jnp.int32)]
            + [pltpu.VMEM((CHUNK, D_MODEL), jnp.float32) for _ in range(RING)]
            + [pltpu.SemaphoreType.DMA for _ in range(2 * RING)]
        ),
    )(x, table)
    return out.reshape(B_ROWS, SEQ, D_MODEL)


# speedup vs baseline: 1.0159x; 1.0158x over previous
import functools
import math

import jax
import jax.numpy as jnp
from jax import lax
from jax.experimental import pallas as pl
from jax.experimental.pallas import tpu as pltpu
from jax.experimental.pallas import tpu_sc as plsc

D_MODEL = 128
SCALE = math.sqrt(float(D_MODEL))

NUM_CORES = 2
NUM_SUBCORES = 16
NUM_WORKERS = NUM_CORES * NUM_SUBCORES
LANES = 16

B_TOTAL = 1024 * 200
B_PER_W = B_TOTAL // NUM_WORKERS
IDX_COLS = 128
IDX_ROWS_PER_W = B_PER_W // IDX_COLS

CHUNK = 128
NCHUNK = B_PER_W // CHUNK
RING = 5
AHEAD = 3
UNROLL_ROWS = 4


def _emb_kernel(idx_hbm, table_hbm, out_hbm, idx_v, *rest):
    bufs = rest[0:RING]
    gsems = rest[RING:2 * RING]
    wsems = rest[2 * RING:3 * RING]

    wid = lax.axis_index("s") * NUM_CORES + lax.axis_index("c")

    pltpu.sync_copy(idx_hbm.at[wid], idx_v)

    out_chunk0 = wid * NCHUNK

    def gather(g, b):
        return pltpu.make_async_copy(table_hbm.at[idx_v.at[g]], bufs[b],
                                     gsems[b])

    def write(g, b):
        row0 = (out_chunk0 + g) * CHUNK
        return pltpu.make_async_copy(bufs[b], out_hbm.at[pl.ds(row0, CHUNK)],
                                     wsems[b])

    def scale(buf):
        def row_body(i, c):
            for r in range(UNROLL_ROWS):
                row = i * UNROLL_ROWS + r
                for j in range(D_MODEL // LANES):
                    sl = pl.ds(j * LANES, LANES)
                    buf[row, sl] = buf[row, sl] * SCALE
            return c

        lax.fori_loop(0, CHUNK // UNROLL_ROWS, row_body, 0)

    def chunk_body(g, slot, with_start, with_drain):
        nslot = (slot + AHEAD) % RING
        if with_drain:
            write(g - (RING - AHEAD), nslot).wait()
        if with_start:
            gather(g + AHEAD, nslot).start()
        gather(g, slot).wait()
        scale(bufs[slot])
        write(g, slot).start()

    for c in range(AHEAD):
        gather(c, c % RING).start()

    for g in range(RING - AHEAD):
        chunk_body(g, g % RING, with_start=True, with_drain=False)

    g0 = RING - AHEAD
    steady = NCHUNK - (RING - AHEAD) - AHEAD
    rounds, leftover = divmod(steady, RING)

    def outer(t, carry):
        for b in range(RING):
            g = g0 + t * RING + b
            chunk_body(g, (g0 + b) % RING, with_start=True, with_drain=True)
        return carry

    lax.fori_loop(0, rounds, outer, 0)

    for i in range(leftover):
        g = g0 + rounds * RING + i
        chunk_body(g, g % RING, with_start=True, with_drain=True)

    for g in range(NCHUNK - AHEAD, NCHUNK):
        chunk_body(g, g % RING, with_start=False, with_drain=False)

    for g in range(NCHUNK - RING, NCHUNK):
        write(g, g % RING).wait()


@functools.partial(jax.jit, static_argnames=())
def kernel(x, table):
    idx3d = x.reshape(NUM_WORKERS, IDX_ROWS_PER_W, IDX_COLS)
    mesh = plsc.VectorSubcoreMesh(core_axis_name="c", subcore_axis_name="s")
    out = pl.kernel(
        _emb_kernel,
        mesh=mesh,
        out_type=jax.ShapeDtypeStruct((B_TOTAL, D_MODEL), jnp.float32),
        scratch_types=(
            [pltpu.VMEM((IDX_ROWS_PER_W, IDX_COLS), jnp.int32)]
            + [pltpu.VMEM((CHUNK, D_MODEL), jnp.float32) for _ in range(RING)]
            + [pltpu.SemaphoreType.DMA for _ in range(2 * RING)]
        ),
    )(idx3d, table)
    return out.reshape(x.shape[0], x.shape[1], D_MODEL)
